# baseline (speedup 1.0000x reference)
import functools

import jax
import jax.numpy as jnp
from jax import lax
from jax.experimental import pallas as pl
from jax.experimental.pallas import tpu as pltpu
from jax.experimental.pallas import tpu_sc as plsc

K_DGCNN = 20
K_LOCAL = 8

_NW = 32
_LANES = 16
_TOPK_CH = 8


def _topk_body(ncols, k, ch, rows_per_w, dist_hbm, out_hbm, row_a, row_b,
               idx_buf, sem_a, sem_b):
    ngroups = 16
    tpg = ncols // (ngroups * _LANES)
    gsz = tpg * _LANES
    wid = lax.axis_index("s") * 2 + lax.axis_index("c")
    row0 = wid * rows_per_w
    lane = lax.iota(jnp.int32, _LANES)
    inf16 = jnp.full((_LANES,), jnp.inf, jnp.float32)
    big16 = jnp.full((_LANES,), 1 << 30, jnp.int32)
    leaf = [lane + t * _LANES for t in range(tpg)]

    def bfly(bv, bi):
        for s in (8, 4, 2, 1):
            perm = lane ^ s
            v2 = bv.at[perm].get(mode='promise_in_bounds')
            i2 = bi.at[perm].get(mode='promise_in_bounds')
            m = (v2 < bv) | ((v2 == bv) & (i2 < bi))
            bv = jnp.where(m, v2, bv)
            bi = jnp.where(m, i2, bi)
        return bv, bi

    def group_min(row_buf, r, base):
        vals = [row_buf[r, pl.ds(base + t * _LANES, _LANES)] for t in range(tpg)]
        idxs = [leaf[t] + base for t in range(tpg)]
        while len(vals) > 1:
            nv, ni = [], []
            for a in range(0, len(vals), 2):
                m = vals[a + 1] < vals[a]
                nv.append(jnp.where(m, vals[a + 1], vals[a]))
                ni.append(jnp.where(m, idxs[a + 1], idxs[a]))
            vals, idxs = nv, ni
        return bfly(vals[0], idxs[0])

    def process_chunk(row_buf, rbase):
        def do_row(r, _):
            def build_g(j, carry):
                cv, ci = carry
                v, i = group_min(row_buf, r, j * gsz)
                m2 = lane == j
                return (jnp.where(m2, v, cv), jnp.where(m2, i, ci))
            cv, ci = lax.fori_loop(0, ngroups, build_g, (inf16, big16))
            acc = [jnp.zeros((_LANES,), jnp.int32) for _ in range((k + 15) // 16)]
            for i in range(k):
                gv, gi = bfly(cv, ci)
                gidx = gi[0]
                sel = lane == (i % 16)
                acc[i // 16] = jnp.where(sel, gi, acc[i // 16])
                if i + 1 < k:
                    woff = (gidx // _LANES) * _LANES
                    wv = row_buf[r, pl.ds(woff, _LANES)]
                    row_buf[r, pl.ds(woff, _LANES)] = jnp.where(
                        lane == (gidx % _LANES), inf16, wv)
                    jw = gidx // gsz
                    v, ii = group_min(row_buf, r, jw * gsz)
                    m2 = lane == jw
                    cv = jnp.where(m2, v, cv)
                    ci = jnp.where(m2, ii, ci)
            for a in range(len(acc)):
                idx_buf[r, pl.ds(a * 16, _LANES)] = acc[a]
            return 0
        lax.fori_loop(0, ch, do_row, 0)
        pltpu.sync_copy(idx_buf, out_hbm.at[pl.ds(rbase, ch)])

    nchunks = rows_per_w // ch
    pltpu.async_copy(dist_hbm.at[pl.ds(row0, ch)], row_a, sem_a)

    def do_pair(p, _):
        ra = row0 + (2 * p) * ch
        rb = ra + ch
        pltpu.async_copy(dist_hbm.at[pl.ds(rb, ch)], row_b, sem_b)
        pltpu.make_async_copy(dist_hbm.at[pl.ds(ra, ch)], row_a, sem_a).wait()
        process_chunk(row_a, ra)

        @pl.when(2 * p + 2 < nchunks)
        def _():
            pltpu.async_copy(dist_hbm.at[pl.ds(rb + ch, ch)], row_a, sem_a)
        pltpu.make_async_copy(dist_hbm.at[pl.ds(rb, ch)], row_b, sem_b).wait()
        process_chunk(row_b, rb)
        return 0
    lax.fori_loop(0, nchunks // 2, do_pair, 0)


def _gr_body(mode, C, K, G, ch, rows_per_w, table_hbm, idx_hbm, *refs):
    if mode == 'edge':
        mx_hbm, s1_hbm, s2_hbm, idx_buf, gath_a, gath_b, mx_b, s1_b, s2_b, sem_a, sem_b = refs
    elif mode == 'gather':
        out_hbm, idx_buf, gath_a, gath_b, sem_a, sem_b = refs
    else:
        sm_hbm, idx_buf, gath_a, gath_b, sm_b, sem_a, sem_b = refs
    wid = lax.axis_index("s") * 2 + lax.axis_index("c")
    row0 = wid * rows_per_w

    def do_chunk(c, _):
        rbase = row0 + c * ch
        pltpu.sync_copy(idx_hbm.at[pl.ds(rbase * K, ch * K)], idx_buf)
        pltpu.async_copy(
            table_hbm.at[idx_buf.at[pl.ds(0, G * K)]], gath_a, sem_a)
        pltpu.async_copy(
            table_hbm.at[idx_buf.at[pl.ds(G * K, G * K)]], gath_b, sem_b)

        def do_group(g, gath, sem):
            pltpu.make_async_copy(
                table_hbm.at[idx_buf.at[pl.ds(0, G * K)]], gath, sem).wait()
            if mode == 'gather':
                pltpu.sync_copy(
                    gath, out_hbm.at[pl.ds((rbase + g * G) * K, G * K)])
                return

            def do_row(rr, _):
                r = g * G + rr
                base = rr * K

                def do_c(cc, _):
                    off = cc * _LANES
                    v = gath[base, pl.ds(off, _LANES)]
                    if mode == 'edge':
                        m = v
                        s = v
                        q = v * v
                        for kk in range(1, K):
                            v = gath[base + kk, pl.ds(off, _LANES)]
                            m = jnp.maximum(m, v)
                            s = s + v
                            q = q + v * v
                        mx_b[r, pl.ds(off, _LANES)] = m
                        s1_b[r, pl.ds(off, _LANES)] = s
                        s2_b[r, pl.ds(off, _LANES)] = q
                    else:
                        s = v
                        for kk in range(1, K):
                            s = s + gath[base + kk, pl.ds(off, _LANES)]
                        sm_b[r, pl.ds(off, _LANES)] = s * (1.0 / K)
                    return 0
                lax.fori_loop(0, C // _LANES, do_c, 0)
                return 0
            lax.fori_loop(0, G, do_row, 0)

        do_group(0, gath_a, sem_a)
        do_group(1, gath_b, sem_b)
        if mode == 'edge':
            pltpu.sync_copy(mx_b, mx_hbm.at[pl.ds(rbase, ch)])
            pltpu.sync_copy(s1_b, s1_hbm.at[pl.ds(rbase, ch)])
            pltpu.sync_copy(s2_b, s2_hbm.at[pl.ds(rbase, ch)])
        elif mode == 'mean':
            pltpu.sync_copy(sm_b, sm_hbm.at[pl.ds(rbase, ch)])
        return 0
    lax.fori_loop(0, rows_per_w // ch, do_chunk, 0)


@functools.partial(jax.jit, static_argnums=(2,))
def _sc_gather_reduce(table, idxg, mode):
    V, C = table.shape
    R, K = idxg.shape
    G = 1
    while G * 2 * K <= 128:
        G *= 2
    assert (G * K) % 8 == 0
    ch = 2 * G
    rows_per_w = R // _NW
    assert rows_per_w % ch == 0
    mesh = plsc.VectorSubcoreMesh(core_axis_name="c", subcore_axis_name="s")
    if mode == 'gather':
        out_type = jax.ShapeDtypeStruct((R * K, C), jnp.float32)
        scratch = [
            pltpu.VMEM((ch * K,), jnp.int32),
            pltpu.VMEM((G * K, C), jnp.float32),
            pltpu.VMEM((G * K, C), jnp.float32),
            pltpu.SemaphoreType.DMA,
            pltpu.SemaphoreType.DMA,
        ]
        kern = pl.kernel(
            functools.partial(_gr_body, mode, C, K, G, ch, rows_per_w),
            mesh=mesh, out_type=out_type, scratch_types=scratch)
        return kern(table, idxg.reshape(-1))
    n_out = 3 if mode == 'edge' else 1
    out_type = [jax.ShapeDtypeStruct((R, C), jnp.float32)] * n_out
    scratch = ([
        pltpu.VMEM((ch * K,), jnp.int32),
        pltpu.VMEM((G * K, C), jnp.float32),
        pltpu.VMEM((G * K, C), jnp.float32),
    ] + [pltpu.VMEM((ch, C), jnp.float32)] * n_out
      + [pltpu.SemaphoreType.DMA, pltpu.SemaphoreType.DMA])
    kern = pl.kernel(
        functools.partial(_gr_body, mode, C, K, G, ch, rows_per_w),
        mesh=mesh,
        out_type=out_type if n_out > 1 else out_type[0],
        scratch_types=scratch,
    )
    return kern(table, idxg.reshape(-1))


@functools.partial(jax.jit, static_argnums=(1,))
def _sc_topk_idx(dist, k):
    R, ncols = dist.shape
    rows_per_w = R // _NW
    ch = _TOPK_CH
    kp = ((k + 15) // 16) * 16
    mesh = plsc.VectorSubcoreMesh(core_axis_name="c", subcore_axis_name="s")
    kern = pl.kernel(
        functools.partial(_topk_body, ncols, k, ch, rows_per_w),
        mesh=mesh,
        out_type=jax.ShapeDtypeStruct((R, kp), jnp.int32),
        scratch_types=[
            pltpu.VMEM((ch, ncols), jnp.float32),
            pltpu.VMEM((ch, ncols), jnp.float32),
            pltpu.VMEM((ch, kp), jnp.int32),
            pltpu.SemaphoreType.DMA,
            pltpu.SemaphoreType.DMA,
        ],
    )
    return kern(dist)[:, :k]


def _ref_knn(x, k):
    inner = -2.0 * jnp.matmul(jnp.transpose(x, (0, 2, 1)), x)
    xx = jnp.sum(x * x, axis=1, keepdims=True)
    dist = xx + inner + jnp.transpose(xx, (0, 2, 1))
    B, N, _ = dist.shape
    return _sc_topk_idx(dist.reshape(B * N, N), k).reshape(B, N, k)


def _ref_edge_features(x, k):
    B, D, N = x.shape
    idx = _ref_knn(x, k)
    x_t = jnp.transpose(x, (0, 2, 1))
    Dp = ((D + 127) // 128) * 128
    tab = x_t.reshape(B * N, D)
    if Dp != D:
        tab = jnp.pad(tab, ((0, 0), (0, Dp - D)))
    idxg = (idx + (jnp.arange(B, dtype=jnp.int32) * N)[:, None, None]).reshape(B * N, k)
    neighbors = _sc_gather_reduce(tab, idxg, 'gather').reshape(B, N, k, Dp)[..., :D]
    center = jnp.broadcast_to(x_t[:, :, None, :], (B, N, k, D))
    feat = jnp.concatenate([neighbors - center, center], axis=3)
    return jnp.transpose(feat, (0, 3, 1, 2))


def _ref_group_norm(x, groups, w, b, eps=1e-5):
    shp = x.shape
    B, C = shp[0], shp[1]
    xg = x.reshape(B, groups, C // groups, -1)
    m = jnp.mean(xg, axis=(2, 3), keepdims=True)
    v = jnp.var(xg, axis=(2, 3), keepdims=True)
    xg = (xg - m) / jnp.sqrt(v + eps)
    xn = xg.reshape(shp)
    wshape = (1, C) + (1,) * (len(shp) - 2)
    return xn * w.reshape(wshape) + b.reshape(wshape)


def _ref_edge_layer(x, W, gw, gb, groups):
    f = _ref_edge_features(x, K_DGCNN)
    y = jnp.einsum('oi,binm->bonm', W, f)
    z = _ref_group_norm(y, groups, gw, gb)
    return jnp.max(jnp.where(z >= 0, z, 0.2 * z), axis=-1)


def _knn_idx(x_t, k):
    xx = jnp.sum(x_t * x_t, axis=2)
    inner = -2.0 * jnp.matmul(x_t, jnp.transpose(x_t, (0, 2, 1)))
    dist = xx[:, :, None] + inner + xx[:, None, :]
    return jax.lax.top_k(-dist, k)[1]


def _gather_reduce(u_t, idx):
    B, N, C = u_t.shape
    g = jax.vmap(lambda t, i: t[i])(u_t, idx.reshape(B, -1))
    g = g.reshape(B, N, -1, C)
    return jnp.max(g, axis=2), jnp.sum(g, axis=2), jnp.sum(g * g, axis=2)


def _edge_layer(x_t, idx, W, gw, gb, groups, eps=1e-5):
    B, N, D = x_t.shape
    O = W.shape[0]
    K = K_DGCNN
    Wa = W[:, :D]
    Wb = W[:, D:]
    u_t = x_t @ Wa.T
    v_t = x_t @ (Wb - Wa).T
    idxg = (idx + (jnp.arange(B, dtype=jnp.int32) * N)[:, None, None]).reshape(B * N, K)
    mx, s1, s2 = _sc_gather_reduce(u_t.reshape(B * N, O), idxg, 'edge')
    mx = mx.reshape(B, N, O)
    s1 = s1.reshape(B, N, O)
    s2 = s2.reshape(B, N, O)
    sum_o = jnp.sum(s1, axis=1) + K * jnp.sum(v_t, axis=1)
    sumsq_o = jnp.sum(s2 + 2.0 * v_t * s1 + K * v_t * v_t, axis=1)
    cnt = (O // groups) * N * K
    sum_g = jnp.sum(sum_o.reshape(B, groups, -1), axis=2)
    sumsq_g = jnp.sum(sumsq_o.reshape(B, groups, -1), axis=2)
    mean_g = sum_g / cnt
    var_g = sumsq_g / cnt - mean_g * mean_g
    inv_g = 1.0 / jnp.sqrt(var_g + eps)
    mean_o = jnp.repeat(mean_g, O // groups, axis=1)
    inv_o = jnp.repeat(inv_g, O // groups, axis=1)
    y_max = mx + v_t
    z = (y_max - mean_o[:, None, :]) * inv_o[:, None, :] * gw[None, None, :] + gb[None, None, :]
    return jnp.where(z >= 0, z, 0.2 * z)


def _gn_rows(y, groups, gw, gb, eps=1e-5):
    B, N, O = y.shape
    yg = y.reshape(B, N, groups, O // groups)
    m = jnp.mean(yg, axis=(1, 3), keepdims=True)
    v = jnp.var(yg, axis=(1, 3), keepdims=True)
    z = ((yg - m) / jnp.sqrt(v + eps)).reshape(B, N, O)
    return z * gw[None, None, :] + gb[None, None, :]


def _lrelu(x):
    return jnp.where(x >= 0, x, 0.2 * x)


def _decoder_body(ni_ref, w1_ref, b1_ref, w2_ref, b2_ref, w3_ref, b3_ref,
                  wm1_ref, bm1_ref, wm2_ref, bm2_ref, wm3_ref, bm3_ref,
                  disp_ref, mat_ref):
    ni = ni_ref[...]
    h = jnp.maximum(jnp.dot(ni, w1_ref[...], preferred_element_type=jnp.float32) + b1_ref[...], 0.0)
    h = jnp.maximum(jnp.dot(h, w2_ref[...], preferred_element_type=jnp.float32) + b2_ref[...], 0.0)
    disp_ref[...] = jnp.dot(h, w3_ref[...], preferred_element_type=jnp.float32) + b3_ref[...]
    hm = jnp.maximum(jnp.dot(ni, wm1_ref[...], preferred_element_type=jnp.float32) + bm1_ref[...], 0.0)
    hm = jnp.maximum(jnp.dot(hm, wm2_ref[...], preferred_element_type=jnp.float32) + bm2_ref[...], 0.0)
    logit = jnp.dot(hm, wm3_ref[...], preferred_element_type=jnp.float32) + bm3_ref[...]
    mat_ref[...] = jax.nn.sigmoid(logit)


def _decoder(node_input, D1, d1b, D2, d2b, D3, d3b, M1, m1b, M2, m2b, M3, m3b):
    B, T, F = node_input.shape
    R = B * T
    FP = 896
    ni = jnp.zeros((R, FP), jnp.float32).at[:, :F].set(node_input.reshape(R, F))
    w1 = jnp.zeros((FP, 256), jnp.float32).at[:F, :].set(D1.T)
    wm1 = jnp.zeros((FP, 128), jnp.float32).at[:F, :].set(M1.T)
    w3 = jnp.zeros((256, 128), jnp.float32).at[:, :3].set(D3.T)
    wm3 = jnp.zeros((64, 128), jnp.float32).at[:, :1].set(M3.T)
    b3 = jnp.zeros((128,), jnp.float32).at[:3].set(d3b)
    bm3 = jnp.zeros((128,), jnp.float32).at[:1].set(m3b)
    RT = 1024
    grid = (R // RT,)
    disp_p, mat_p = pl.pallas_call(
        _decoder_body,
        grid=grid,
        in_specs=[
            pl.BlockSpec((RT, FP), lambda i: (i, 0)),
            pl.BlockSpec((FP, 256), lambda i: (0, 0)),
            pl.BlockSpec((256,), lambda i: (0,)),
            pl.BlockSpec((256, 256), lambda i: (0, 0)),
            pl.BlockSpec((256,), lambda i: (0,)),
            pl.BlockSpec((256, 128), lambda i: (0, 0)),
            pl.BlockSpec((128,), lambda i: (0,)),
            pl.BlockSpec((FP, 128), lambda i: (0, 0)),
            pl.BlockSpec((128,), lambda i: (0,)),
            pl.BlockSpec((128, 64), lambda i: (0, 0)),
            pl.BlockSpec((64,), lambda i: (0,)),
            pl.BlockSpec((64, 128), lambda i: (0, 0)),
            pl.BlockSpec((128,), lambda i: (0,)),
        ],
        out_specs=[
            pl.BlockSpec((RT, 128), lambda i: (i, 0)),
            pl.BlockSpec((RT, 128), lambda i: (i, 0)),
        ],
        out_shape=[
            jax.ShapeDtypeStruct((R, 128), jnp.float32),
            jax.ShapeDtypeStruct((R, 128), jnp.float32),
        ],
    )(ni, w1, d1b, jnp.asarray(D2.T), d2b, w3, b3, wm1, m1b,
      jnp.asarray(M2.T), m2b, wm3, bm3)
    disp = disp_p[:, :3].reshape(B, T, 3)
    mat = mat_p[:, 0].reshape(B, T)
    return disp, mat


def kernel(surface, template, W1, g1w, g1b, W2, g2w, g2b, W3, g3w, g3b,
           Wp, gpw, gpb, Wg, ggw, ggb,
           D1, d1b, D2, d2b, D3, d3b, M1, m1b, M2, m2b, M3, m3b):
    B, S, _ = surface.shape
    T = template.shape[1]
    x = jnp.transpose(surface, (0, 2, 1))
    x1 = _ref_edge_layer(x, W1, g1w, g1b, 8)
    x2 = _ref_edge_layer(x1, W2, g2w, g2b, 8)
    idx3 = _ref_knn(x2, K_DGCNN)
    x2_t = jnp.transpose(x2, (0, 2, 1))
    x3 = _edge_layer(x2_t, idx3, W3, g3w, g3b, 16)
    cat = jnp.concatenate(
        [jnp.transpose(x1, (0, 2, 1)), x2_t, x3], axis=2)
    point_feat = _lrelu(_gn_rows(cat @ Wp.T, 16, gpw, gpb))
    g = _lrelu(_gn_rows(cat @ Wg.T, 16, ggw, ggb))
    global_feat = jnp.concatenate([jnp.max(g, axis=1), jnp.mean(g, axis=1)], axis=1)

    surf_xyz = surface[:, :, :3]
    s2 = jnp.sum(surf_xyz * surf_xyz, axis=2)
    t2 = jnp.sum(template * template, axis=2)
    d2 = t2[:, :, None] + s2[:, None, :] - 2.0 * jnp.matmul(template, jnp.transpose(surf_xyz, (0, 2, 1)))
    nn_idx = _sc_topk_idx(d2.reshape(B * T, S), K_LOCAL).reshape(B, T, K_LOCAL)
    nn_g = (nn_idx + (jnp.arange(B, dtype=jnp.int32) * S)[:, None, None]).reshape(B * T, K_LOCAL)
    Dp = point_feat.shape[2]
    local_feat = _sc_gather_reduce(point_feat.reshape(B * S, Dp), nn_g, 'mean').reshape(B, T, Dp)

    global_exp = jnp.broadcast_to(global_feat[:, None, :], (B, T, global_feat.shape[1]))
    node_input = jnp.concatenate([template, local_feat, global_exp], axis=2)
    return _decoder(node_input, D1, d1b, D2, d2b, D3, d3b, M1, m1b, M2, m2b, M3, m3b)


# speedup vs baseline: 1.0864x; 1.0864x over previous
import functools

import jax
import jax.numpy as jnp
from jax import lax
from jax.experimental import pallas as pl
from jax.experimental.pallas import tpu as pltpu
from jax.experimental.pallas import tpu_sc as plsc

K_DGCNN = 20
K_LOCAL = 8

_NW = 32
_LANES = 16
_TOPK_CH = 8


def _topk_body(ncols, k, ch, rows_per_w, dist_hbm, out_hbm, row_a, row_b,
               idx_buf, sem_a, sem_b):
    ngroups = 16
    tpg = ncols // (ngroups * _LANES)
    gsz = tpg * _LANES
    wid = lax.axis_index("s") * 2 + lax.axis_index("c")
    row0 = wid * rows_per_w
    lane = lax.iota(jnp.int32, _LANES)
    inf16 = jnp.full((_LANES,), jnp.inf, jnp.float32)
    big16 = jnp.full((_LANES,), 1 << 30, jnp.int32)
    leaf = [lane + t * _LANES for t in range(tpg)]

    def bfly(bv, bi):
        for s in (8, 4, 2, 1):
            perm = lane ^ s
            v2 = bv.at[perm].get(mode='promise_in_bounds')
            i2 = bi.at[perm].get(mode='promise_in_bounds')
            m = (v2 < bv) | ((v2 == bv) & (i2 < bi))
            bv = jnp.where(m, v2, bv)
            bi = jnp.where(m, i2, bi)
        return bv, bi

    def group_min(row_buf, r, base):
        vals = [row_buf[r, pl.ds(base + t * _LANES, _LANES)] for t in range(tpg)]
        idxs = [leaf[t] + base for t in range(tpg)]
        while len(vals) > 1:
            nv, ni = [], []
            for a in range(0, len(vals), 2):
                m = vals[a + 1] < vals[a]
                nv.append(jnp.where(m, vals[a + 1], vals[a]))
                ni.append(jnp.where(m, idxs[a + 1], idxs[a]))
            vals, idxs = nv, ni
        return bfly(vals[0], idxs[0])

    def process_chunk(row_buf, rbase):
        def do_row(r, _):
            def build_g(j, carry):
                cv, ci = carry
                v, i = group_min(row_buf, r, j * gsz)
                m2 = lane == j
                return (jnp.where(m2, v, cv), jnp.where(m2, i, ci))
            cv, ci = lax.fori_loop(0, ngroups, build_g, (inf16, big16))
            acc = [jnp.zeros((_LANES,), jnp.int32) for _ in range((k + 15) // 16)]
            for i in range(k):
                gv, gi = bfly(cv, ci)
                gidx = gi[0]
                sel = lane == (i % 16)
                acc[i // 16] = jnp.where(sel, gi, acc[i // 16])
                if i + 1 < k:
                    woff = (gidx // _LANES) * _LANES
                    wv = row_buf[r, pl.ds(woff, _LANES)]
                    row_buf[r, pl.ds(woff, _LANES)] = jnp.where(
                        lane == (gidx % _LANES), inf16, wv)
                    jw = gidx // gsz
                    v, ii = group_min(row_buf, r, jw * gsz)
                    m2 = lane == jw
                    cv = jnp.where(m2, v, cv)
                    ci = jnp.where(m2, ii, ci)
            for a in range(len(acc)):
                idx_buf[r, pl.ds(a * 16, _LANES)] = acc[a]
            return 0
        lax.fori_loop(0, ch, do_row, 0)
        pltpu.sync_copy(idx_buf, out_hbm.at[pl.ds(rbase, ch)])

    def do_chunk(c, _):
        rbase = row0 + c * ch
        pltpu.sync_copy(dist_hbm.at[pl.ds(rbase, ch)], row_a)
        process_chunk(row_a, rbase)
        return 0
    lax.fori_loop(0, rows_per_w // ch, do_chunk, 0)


def _gr_body(mode, C, K, G, ch, rows_per_w, table_hbm, idx_hbm, *refs):
    if mode == 'edge':
        mx_hbm, s1_hbm, s2_hbm, idx_buf, gath_a, gath_b, mx_b, s1_b, s2_b, sem_a, sem_b = refs
    elif mode == 'gather':
        out_hbm, idx_buf, gath_a, gath_b, sem_a, sem_b = refs
    else:
        sm_hbm, idx_buf, gath_a, gath_b, sm_b, sem_a, sem_b = refs
    wid = lax.axis_index("s") * 2 + lax.axis_index("c")
    row0 = wid * rows_per_w

    def do_chunk(c, _):
        rbase = row0 + c * ch
        pltpu.sync_copy(idx_hbm.at[pl.ds(rbase * K, ch * K)], idx_buf)
        pltpu.async_copy(
            table_hbm.at[idx_buf.at[pl.ds(0, G * K)]], gath_a, sem_a)
        pltpu.async_copy(
            table_hbm.at[idx_buf.at[pl.ds(G * K, G * K)]], gath_b, sem_b)

        def do_group(g, gath, sem):
            pltpu.make_async_copy(
                table_hbm.at[idx_buf.at[pl.ds(0, G * K)]], gath, sem).wait()
            if mode == 'gather':
                pltpu.sync_copy(
                    gath, out_hbm.at[pl.ds((rbase + g * G) * K, G * K)])
                return

            def do_row(rr, _):
                r = g * G + rr
                base = rr * K

                def do_c(cc, _):
                    off = cc * _LANES
                    v = gath[base, pl.ds(off, _LANES)]
                    if mode == 'edge':
                        m = v
                        s = v
                        q = v * v
                        for kk in range(1, K):
                            v = gath[base + kk, pl.ds(off, _LANES)]
                            m = jnp.maximum(m, v)
                            s = s + v
                            q = q + v * v
                        mx_b[r, pl.ds(off, _LANES)] = m
                        s1_b[r, pl.ds(off, _LANES)] = s
                        s2_b[r, pl.ds(off, _LANES)] = q
                    else:
                        s = v
                        for kk in range(1, K):
                            s = s + gath[base + kk, pl.ds(off, _LANES)]
                        sm_b[r, pl.ds(off, _LANES)] = s * (1.0 / K)
                    return 0
                lax.fori_loop(0, C // _LANES, do_c, 0)
                return 0
            lax.fori_loop(0, G, do_row, 0)

        do_group(0, gath_a, sem_a)
        do_group(1, gath_b, sem_b)
        if mode == 'edge':
            pltpu.sync_copy(mx_b, mx_hbm.at[pl.ds(rbase, ch)])
            pltpu.sync_copy(s1_b, s1_hbm.at[pl.ds(rbase, ch)])
            pltpu.sync_copy(s2_b, s2_hbm.at[pl.ds(rbase, ch)])
        elif mode == 'mean':
            pltpu.sync_copy(sm_b, sm_hbm.at[pl.ds(rbase, ch)])
        return 0
    lax.fori_loop(0, rows_per_w // ch, do_chunk, 0)


@functools.partial(jax.jit, static_argnums=(2,))
def _sc_gather_reduce(table, idxg, mode):
    V, C = table.shape
    R, K = idxg.shape
    G = 1
    while G * 2 * K <= 128:
        G *= 2
    assert (G * K) % 8 == 0
    ch = 2 * G
    rows_per_w = R // _NW
    assert rows_per_w % ch == 0
    mesh = plsc.VectorSubcoreMesh(core_axis_name="c", subcore_axis_name="s")
    if mode == 'gather':
        out_type = jax.ShapeDtypeStruct((R * K, C), jnp.float32)
        scratch = [
            pltpu.VMEM((ch * K,), jnp.int32),
            pltpu.VMEM((G * K, C), jnp.float32),
            pltpu.VMEM((G * K, C), jnp.float32),
            pltpu.SemaphoreType.DMA,
            pltpu.SemaphoreType.DMA,
        ]
        kern = pl.kernel(
            functools.partial(_gr_body, mode, C, K, G, ch, rows_per_w),
            mesh=mesh, out_type=out_type, scratch_types=scratch)
        return kern(table, idxg.reshape(-1))
    n_out = 3 if mode == 'edge' else 1
    out_type = [jax.ShapeDtypeStruct((R, C), jnp.float32)] * n_out
    scratch = ([
        pltpu.VMEM((ch * K,), jnp.int32),
        pltpu.VMEM((G * K, C), jnp.float32),
        pltpu.VMEM((G * K, C), jnp.float32),
    ] + [pltpu.VMEM((ch, C), jnp.float32)] * n_out
      + [pltpu.SemaphoreType.DMA, pltpu.SemaphoreType.DMA])
    kern = pl.kernel(
        functools.partial(_gr_body, mode, C, K, G, ch, rows_per_w),
        mesh=mesh,
        out_type=out_type if n_out > 1 else out_type[0],
        scratch_types=scratch,
    )
    return kern(table, idxg.reshape(-1))


@functools.partial(jax.jit, static_argnums=(1,))
def _sc_topk_idx(dist, k):
    R, ncols = dist.shape
    rows_per_w = R // _NW
    ch = _TOPK_CH
    kp = ((k + 15) // 16) * 16
    mesh = plsc.VectorSubcoreMesh(core_axis_name="c", subcore_axis_name="s")
    kern = pl.kernel(
        functools.partial(_topk_body, ncols, k, ch, rows_per_w),
        mesh=mesh,
        out_type=jax.ShapeDtypeStruct((R, kp), jnp.int32),
        scratch_types=[
            pltpu.VMEM((ch, ncols), jnp.float32),
            pltpu.VMEM((ch, ncols), jnp.float32),
            pltpu.VMEM((ch, kp), jnp.int32),
            pltpu.SemaphoreType.DMA,
            pltpu.SemaphoreType.DMA,
        ],
    )
    return kern(dist)[:, :k]


def _ref_knn(x, k):
    inner = -2.0 * jnp.matmul(jnp.transpose(x, (0, 2, 1)), x)
    xx = jnp.sum(x * x, axis=1, keepdims=True)
    dist = xx + inner + jnp.transpose(xx, (0, 2, 1))
    B, N, _ = dist.shape
    return _sc_topk_idx(dist.reshape(B * N, N), k).reshape(B, N, k)


def _ref_edge_features(x, k):
    B, D, N = x.shape
    idx = _ref_knn(x, k)
    x_t = jnp.transpose(x, (0, 2, 1))
    Dp = ((D + 127) // 128) * 128
    tab = x_t.reshape(B * N, D)
    if Dp != D:
        tab = jnp.pad(tab, ((0, 0), (0, Dp - D)))
    idxg = (idx + (jnp.arange(B, dtype=jnp.int32) * N)[:, None, None]).reshape(B * N, k)
    neighbors = _sc_gather_reduce(tab, idxg, 'gather').reshape(B, N, k, Dp)[..., :D]
    center = jnp.broadcast_to(x_t[:, :, None, :], (B, N, k, D))
    feat = jnp.concatenate([neighbors - center, center], axis=3)
    return jnp.transpose(feat, (0, 3, 1, 2))


def _ref_group_norm(x, groups, w, b, eps=1e-5):
    shp = x.shape
    B, C = shp[0], shp[1]
    xg = x.reshape(B, groups, C // groups, -1)
    m = jnp.mean(xg, axis=(2, 3), keepdims=True)
    v = jnp.var(xg, axis=(2, 3), keepdims=True)
    xg = (xg - m) / jnp.sqrt(v + eps)
    xn = xg.reshape(shp)
    wshape = (1, C) + (1,) * (len(shp) - 2)
    return xn * w.reshape(wshape) + b.reshape(wshape)


def _ref_edge_layer(x, W, gw, gb, groups):
    f = _ref_edge_features(x, K_DGCNN)
    y = jnp.einsum('oi,binm->bonm', W, f)
    z = _ref_group_norm(y, groups, gw, gb)
    return jnp.max(jnp.where(z >= 0, z, 0.2 * z), axis=-1)


def _knn_idx(x_t, k):
    xx = jnp.sum(x_t * x_t, axis=2)
    inner = -2.0 * jnp.matmul(x_t, jnp.transpose(x_t, (0, 2, 1)))
    dist = xx[:, :, None] + inner + xx[:, None, :]
    return jax.lax.top_k(-dist, k)[1]


def _gather_reduce(u_t, idx):
    B, N, C = u_t.shape
    g = jax.vmap(lambda t, i: t[i])(u_t, idx.reshape(B, -1))
    g = g.reshape(B, N, -1, C)
    return jnp.max(g, axis=2), jnp.sum(g, axis=2), jnp.sum(g * g, axis=2)


def _edge_layer(x_t, idx, W, gw, gb, groups, eps=1e-5):
    B, N, D = x_t.shape
    O = W.shape[0]
    K = K_DGCNN
    Wa = W[:, :D]
    Wb = W[:, D:]
    u_t = x_t @ Wa.T
    v_t = x_t @ (Wb - Wa).T
    idxg = (idx + (jnp.arange(B, dtype=jnp.int32) * N)[:, None, None]).reshape(B * N, K)
    mx, s1, s2 = _sc_gather_reduce(u_t.reshape(B * N, O), idxg, 'edge')
    mx = mx.reshape(B, N, O)
    s1 = s1.reshape(B, N, O)
    s2 = s2.reshape(B, N, O)
    sum_o = jnp.sum(s1, axis=1) + K * jnp.sum(v_t, axis=1)
    sumsq_o = jnp.sum(s2 + 2.0 * v_t * s1 + K * v_t * v_t, axis=1)
    cnt = (O // groups) * N * K
    sum_g = jnp.sum(sum_o.reshape(B, groups, -1), axis=2)
    sumsq_g = jnp.sum(sumsq_o.reshape(B, groups, -1), axis=2)
    mean_g = sum_g / cnt
    var_g = sumsq_g / cnt - mean_g * mean_g
    inv_g = 1.0 / jnp.sqrt(var_g + eps)
    mean_o = jnp.repeat(mean_g, O // groups, axis=1)
    inv_o = jnp.repeat(inv_g, O // groups, axis=1)
    y_max = mx + v_t
    z = (y_max - mean_o[:, None, :]) * inv_o[:, None, :] * gw[None, None, :] + gb[None, None, :]
    return jnp.where(z >= 0, z, 0.2 * z)


def _gn_rows(y, groups, gw, gb, eps=1e-5):
    B, N, O = y.shape
    yg = y.reshape(B, N, groups, O // groups)
    m = jnp.mean(yg, axis=(1, 3), keepdims=True)
    v = jnp.var(yg, axis=(1, 3), keepdims=True)
    z = ((yg - m) / jnp.sqrt(v + eps)).reshape(B, N, O)
    return z * gw[None, None, :] + gb[None, None, :]


def _lrelu(x):
    return jnp.where(x >= 0, x, 0.2 * x)


def _decoder_body(ni_ref, w1_ref, b1_ref, w2_ref, b2_ref, w3_ref, b3_ref,
                  wm1_ref, bm1_ref, wm2_ref, bm2_ref, wm3_ref, bm3_ref,
                  disp_ref, mat_ref):
    ni = ni_ref[...]
    h = jnp.maximum(jnp.dot(ni, w1_ref[...], preferred_element_type=jnp.float32) + b1_ref[...], 0.0)
    h = jnp.maximum(jnp.dot(h, w2_ref[...], preferred_element_type=jnp.float32) + b2_ref[...], 0.0)
    disp_ref[...] = jnp.dot(h, w3_ref[...], preferred_element_type=jnp.float32) + b3_ref[...]
    hm = jnp.maximum(jnp.dot(ni, wm1_ref[...], preferred_element_type=jnp.float32) + bm1_ref[...], 0.0)
    hm = jnp.maximum(jnp.dot(hm, wm2_ref[...], preferred_element_type=jnp.float32) + bm2_ref[...], 0.0)
    logit = jnp.dot(hm, wm3_ref[...], preferred_element_type=jnp.float32) + bm3_ref[...]
    mat_ref[...] = jax.nn.sigmoid(logit)


def _decoder(node_input, D1, d1b, D2, d2b, D3, d3b, M1, m1b, M2, m2b, M3, m3b):
    B, T, F = node_input.shape
    R = B * T
    FP = 896
    ni = jnp.zeros((R, FP), jnp.float32).at[:, :F].set(node_input.reshape(R, F))
    w1 = jnp.zeros((FP, 256), jnp.float32).at[:F, :].set(D1.T)
    wm1 = jnp.zeros((FP, 128), jnp.float32).at[:F, :].set(M1.T)
    w3 = jnp.zeros((256, 128), jnp.float32).at[:, :3].set(D3.T)
    wm3 = jnp.zeros((64, 128), jnp.float32).at[:, :1].set(M3.T)
    b3 = jnp.zeros((128,), jnp.float32).at[:3].set(d3b)
    bm3 = jnp.zeros((128,), jnp.float32).at[:1].set(m3b)
    RT = 1024
    grid = (R // RT,)
    disp_p, mat_p = pl.pallas_call(
        _decoder_body,
        grid=grid,
        in_specs=[
            pl.BlockSpec((RT, FP), lambda i: (i, 0)),
            pl.BlockSpec((FP, 256), lambda i: (0, 0)),
            pl.BlockSpec((256,), lambda i: (0,)),
            pl.BlockSpec((256, 256), lambda i: (0, 0)),
            pl.BlockSpec((256,), lambda i: (0,)),
            pl.BlockSpec((256, 128), lambda i: (0, 0)),
            pl.BlockSpec((128,), lambda i: (0,)),
            pl.BlockSpec((FP, 128), lambda i: (0, 0)),
            pl.BlockSpec((128,), lambda i: (0,)),
            pl.BlockSpec((128, 64), lambda i: (0, 0)),
            pl.BlockSpec((64,), lambda i: (0,)),
            pl.BlockSpec((64, 128), lambda i: (0, 0)),
            pl.BlockSpec((128,), lambda i: (0,)),
        ],
        out_specs=[
            pl.BlockSpec((RT, 128), lambda i: (i, 0)),
            pl.BlockSpec((RT, 128), lambda i: (i, 0)),
        ],
        out_shape=[
            jax.ShapeDtypeStruct((R, 128), jnp.float32),
            jax.ShapeDtypeStruct((R, 128), jnp.float32),
        ],
    )(ni, w1, d1b, jnp.asarray(D2.T), d2b, w3, b3, wm1, m1b,
      jnp.asarray(M2.T), m2b, wm3, bm3)
    disp = disp_p[:, :3].reshape(B, T, 3)
    mat = mat_p[:, 0].reshape(B, T)
    return disp, mat


def kernel(surface, template, W1, g1w, g1b, W2, g2w, g2b, W3, g3w, g3b,
           Wp, gpw, gpb, Wg, ggw, ggb,
           D1, d1b, D2, d2b, D3, d3b, M1, m1b, M2, m2b, M3, m3b):
    B, S, _ = surface.shape
    T = template.shape[1]
    x = jnp.transpose(surface, (0, 2, 1))
    x1 = _ref_edge_layer(x, W1, g1w, g1b, 8)
    x2 = _ref_edge_layer(x1, W2, g2w, g2b, 8)
    idx3 = _ref_knn(x2, K_DGCNN)
    x2_t = jnp.transpose(x2, (0, 2, 1))
    x3 = _edge_layer(x2_t, idx3, W3, g3w, g3b, 16)
    cat = jnp.concatenate(
        [jnp.transpose(x1, (0, 2, 1)), x2_t, x3], axis=2)
    point_feat = _lrelu(_gn_rows(cat @ Wp.T, 16, gpw, gpb))
    g = _lrelu(_gn_rows(cat @ Wg.T, 16, ggw, ggb))
    global_feat = jnp.concatenate([jnp.max(g, axis=1), jnp.mean(g, axis=1)], axis=1)

    surf_xyz = surface[:, :, :3]
    s2 = jnp.sum(surf_xyz * surf_xyz, axis=2)
    t2 = jnp.sum(template * template, axis=2)
    d2 = t2[:, :, None] + s2[:, None, :] - 2.0 * jnp.matmul(template, jnp.transpose(surf_xyz, (0, 2, 1)))
    nn_idx = _sc_topk_idx(d2.reshape(B * T, S), K_LOCAL).reshape(B, T, K_LOCAL)
    nn_g = (nn_idx + (jnp.arange(B, dtype=jnp.int32) * S)[:, None, None]).reshape(B * T, K_LOCAL)
    Dp = point_feat.shape[2]
    local_feat = _sc_gather_reduce(point_feat.reshape(B * S, Dp), nn_g, 'mean').reshape(B, T, Dp)

    global_exp = jnp.broadcast_to(global_feat[:, None, :], (B, T, global_feat.shape[1]))
    node_input = jnp.concatenate([template, local_feat, global_exp], axis=2)
    return _decoder(node_input, D1, d1b, D2, d2b, D3, d3b, M1, m1b, M2, m2b, M3, m3b)
